# baseline (speedup 1.0000x reference)
import functools

import jax
import jax.numpy as jnp
from jax import lax
from jax.experimental import pallas as pl
from jax.experimental.pallas import tpu as pltpu
from jax.experimental.pallas import tpu_sc as plsc

N = 10000
D = 128
NC = 2
NS = 16
NW = NC * NS
NPAD = 10240
STRIPE = NPAD // NS


_HROWS = NPAD // 128
_HSTRIPE = _HROWS // NS


def _deg_body(d1_hbm, zer_hbm, out_hbm, di_v, h1_v, h2_v, idx_v, acc_s):
    c = lax.axis_index("c")
    s = lax.axis_index("s")
    w = c * NS + s
    ew = di_v.shape[0]
    pltpu.sync_copy(d1_hbm.at[pl.ds(w * ew, ew)], di_v)

    zv = jnp.zeros((16,), jnp.float32)

    def zfill(i, _):
        h1_v[pl.ds(i * 16, 16)] = zv
        return 0

    lax.fori_loop(0, NPAD // 16, zfill, 0)

    def ifill(k, _):
        idx_v[pl.ds(k * 16, 16)] = lax.iota(jnp.int32, 16) + k * 16
        return 0

    lax.fori_loop(0, _HROWS // 16, ifill, 0)

    @pl.when(s < _HROWS // 8)
    def _():
        pltpu.sync_copy(zer_hbm.at[pl.ds(0, 8)],
                        acc_s.at[pl.ds(s * 8, 8)])

    ones = jnp.full((16,), 1.0, jnp.float32)

    def hadd(i, _):
        dv = di_v[pl.ds(i * 16, 16)]
        plsc.addupdate_scatter(h1_v, [dv], ones)
        return 0

    lax.fori_loop(0, ew // 16, hadd, 0)

    def rsh(i, _):
        r = i // 8
        k = i - r * 8
        h2_v[r, pl.ds(k * 16, 16)] = h1_v[pl.ds(i * 16, 16)]
        return 0

    lax.fori_loop(0, NPAD // 16, rsh, 0)
    plsc.subcore_barrier()
    pltpu.sync_copy(h2_v, acc_s.at[idx_v], add=True)
    plsc.subcore_barrier()

    @pl.when(s < _HROWS // 8)
    def _():
        pltpu.sync_copy(acc_s.at[pl.ds(s * 8, 8)],
                        out_hbm.at[c].at[pl.ds(s * 8, 8)])


def _sc_degree(d1, zer):
    ew = d1.shape[0] // NW
    mesh = plsc.VectorSubcoreMesh(core_axis_name="c", subcore_axis_name="s")
    f = pl.kernel(
        _deg_body,
        out_type=jax.ShapeDtypeStruct((NC, _HROWS, 128), jnp.float32),
        mesh=mesh,
        scratch_types=[
            pltpu.VMEM((ew,), jnp.int32),
            pltpu.VMEM((NPAD,), jnp.float32),
            pltpu.VMEM((_HROWS, 128), jnp.float32),
            pltpu.VMEM((_HROWS,), jnp.int32),
            pltpu.VMEM_SHARED((_HROWS, 128), jnp.float32),
        ],
        compiler_params=pltpu.CompilerParams(needs_layout_passes=False),
    )
    return f(d1, zer)


def _agg_body(h_hbm, si_hbm, di_hbm, out0_hbm, out1_hbm,
              si_v, di_v, rows_v, rb2_v, zb_v, acc_s, sem, sem2):
    c = lax.axis_index("c")
    s = lax.axis_index("s")
    w = c * NS + s
    nchunks = si_hbm.shape[1]
    nstage = si_v.shape[0]

    pltpu.async_copy(si_hbm.at[w].at[pl.ds(0, nstage)], si_v, sem)
    pltpu.async_copy(di_hbm.at[w].at[pl.ds(0, nstage)], di_v, sem2)

    @pl.when(c == 0)
    def _():
        pltpu.sync_copy(h_hbm.at[pl.ds(s * STRIPE, STRIPE)],
                        acc_s.at[pl.ds(s * STRIPE, STRIPE)])

    @pl.when(c != 0)
    def _():
        zv = jnp.zeros((16,), jnp.float32)
        zrows = zb_v.shape[0]

        def zfill(i, _):
            r = i // 8
            k = i - r * 8
            zb_v[r, pl.ds(k * 16, 16)] = zv
            return 0

        lax.fori_loop(0, zrows * 8, zfill, 0)
        for k in range(STRIPE // zrows):
            pltpu.sync_copy(zb_v,
                            acc_s.at[pl.ds(s * STRIPE + k * zrows, zrows)])

    pltpu.make_async_copy(si_hbm.at[w].at[pl.ds(0, nstage)], si_v, sem).wait()
    pltpu.make_async_copy(di_hbm.at[w].at[pl.ds(0, nstage)], di_v, sem2).wait()
    plsc.subcore_barrier()

    for p in range(nchunks // nstage):
        if p > 0:
            pltpu.sync_copy(si_hbm.at[w].at[pl.ds(p * nstage, nstage)], si_v)
            pltpu.sync_copy(di_hbm.at[w].at[pl.ds(p * nstage, nstage)], di_v)
        pltpu.sync_copy(h_hbm.at[si_v.at[0]], rows_v)
        pltpu.async_copy(rows_v, acc_s.at[di_v.at[0]], sem, add=True)

        def body(k, _):
            j1 = 2 * k + 1
            j2 = j1 + 1
            pltpu.sync_copy(h_hbm.at[si_v.at[j1]], rb2_v)
            pltpu.make_async_copy(rows_v, acc_s.at[di_v.at[j1 - 1]],
                                  sem).wait()
            pltpu.async_copy(rb2_v, acc_s.at[di_v.at[j1]], sem2, add=True)

            @pl.when(j2 < nstage)
            def _():
                pltpu.sync_copy(h_hbm.at[si_v.at[j2]], rows_v)
                pltpu.make_async_copy(rb2_v, acc_s.at[di_v.at[j1]],
                                      sem2).wait()
                pltpu.async_copy(rows_v, acc_s.at[di_v.at[j2]], sem, add=True)

            @pl.when(j2 >= nstage)
            def _():
                pltpu.make_async_copy(rb2_v, acc_s.at[di_v.at[j1]],
                                      sem2).wait()

            return 0

        lax.fori_loop(0, nstage // 2, body, 0)

    plsc.subcore_barrier()

    @pl.when(c == 0)
    def _():
        pltpu.sync_copy(acc_s.at[pl.ds(s * STRIPE, STRIPE)],
                        out0_hbm.at[pl.ds(s * STRIPE, STRIPE)])

    @pl.when(c != 0)
    def _():
        pltpu.sync_copy(acc_s.at[pl.ds(s * STRIPE, STRIPE)],
                        out1_hbm.at[pl.ds(s * STRIPE, STRIPE)])


def _sc_aggregate(h, si3, di3):
    nchunks = si3.shape[1]
    cw = si3.shape[2]
    mesh = plsc.VectorSubcoreMesh(core_axis_name="c", subcore_axis_name="s")
    f = pl.kernel(
        _agg_body,
        out_type=[jax.ShapeDtypeStruct((NPAD, D), jnp.float32),
                  jax.ShapeDtypeStruct((NPAD, D), jnp.float32)],
        mesh=mesh,
        scratch_types=[
            pltpu.VMEM((nchunks // 2, cw), jnp.int32),
            pltpu.VMEM((nchunks // 2, cw), jnp.int32),
            pltpu.VMEM((cw, D), jnp.float32),
            pltpu.VMEM((cw, D), jnp.float32),
            pltpu.VMEM((40, D), jnp.float32),
            pltpu.VMEM_SHARED((NPAD, D), jnp.float32),
            pltpu.SemaphoreType.DMA,
            pltpu.SemaphoreType.DMA,
        ],
    )
    return f(h, si3, di3)


_BLK = 512
_GRID = NPAD // _BLK


def _mm1_body(d0_ref, d1_ref, x_ref, w_ref, o_ref, dis_ref):
    deg = 1.0 + d0_ref[...] + d1_ref[...]
    dis = lax.rsqrt(deg)
    dis_ref[...] = dis
    o_ref[...] = jnp.dot(x_ref[...], w_ref[...],
                         preferred_element_type=jnp.float32) * dis


def _tc_first(d0, d1, x, w):
    return pl.pallas_call(
        _mm1_body,
        grid=(_GRID,),
        in_specs=[
            pl.BlockSpec((_BLK, 1), lambda i: (i, 0)),
            pl.BlockSpec((_BLK, 1), lambda i: (i, 0)),
            pl.BlockSpec((_BLK, D), lambda i: (i, 0)),
            pl.BlockSpec((D, D), lambda i: (0, 0)),
        ],
        out_specs=[
            pl.BlockSpec((_BLK, D), lambda i: (i, 0)),
            pl.BlockSpec((_BLK, 1), lambda i: (i, 0)),
        ],
        out_shape=[
            jax.ShapeDtypeStruct((NPAD, D), jnp.float32),
            jax.ShapeDtypeStruct((NPAD, 1), jnp.float32),
        ],
    )(d0, d1, x, w)


def _layer_body(p0_ref, p1_ref, dis_ref, b_ref, w_ref, o_ref):
    dis = dis_ref[...]
    t = (p0_ref[...] + p1_ref[...]) * dis + b_ref[...]
    a = jnp.maximum(t, 0.0)
    o_ref[...] = jnp.dot(a, w_ref[...],
                         preferred_element_type=jnp.float32) * dis


def _tc_layer(p, dis, b, w):
    return pl.pallas_call(
        _layer_body,
        grid=(_GRID,),
        in_specs=[
            pl.BlockSpec((_BLK, D), lambda i: (i, 0)),
            pl.BlockSpec((_BLK, D), lambda i: (i, 0)),
            pl.BlockSpec((_BLK, 1), lambda i: (i, 0)),
            pl.BlockSpec((1, D), lambda i: (0, 0)),
            pl.BlockSpec((D, D), lambda i: (0, 0)),
        ],
        out_specs=pl.BlockSpec((_BLK, D), lambda i: (i, 0)),
        out_shape=jax.ShapeDtypeStruct((NPAD, D), jnp.float32),
    )(p[0], p[1], dis, b, w)


def _final_body(p0_ref, p1_ref, dis_ref, b_ref, o_ref):
    t = (p0_ref[...] + p1_ref[...]) * dis_ref[...] + b_ref[...]
    m = jnp.max(t, axis=1, keepdims=True)
    e = jnp.exp(t - m)
    ssum = jnp.sum(e, axis=1, keepdims=True)
    o_ref[...] = t - m - jnp.log(ssum)


def _tc_final(p, dis, b):
    blk = 1000
    return pl.pallas_call(
        _final_body,
        grid=(N // blk,),
        in_specs=[
            pl.BlockSpec((blk, D), lambda i: (i, 0)),
            pl.BlockSpec((blk, D), lambda i: (i, 0)),
            pl.BlockSpec((blk, 1), lambda i: (i, 0)),
            pl.BlockSpec((1, D), lambda i: (0, 0)),
        ],
        out_specs=pl.BlockSpec((blk, D), lambda i: (i, 0)),
        out_shape=jax.ShapeDtypeStruct((N, D), jnp.float32),
    )(p[0], p[1], dis, b)


@jax.jit
def kernel(x, edge_index, W1, b1, W2, b2, W3, b3):
    E = edge_index.shape[1]
    nchunks = 80
    cw = E // (NW * nchunks)

    src = edge_index[0].astype(jnp.int32)
    dst = edge_index[1].astype(jnp.int32)
    si3 = src.reshape(NW, nchunks, cw)
    di3 = dst.reshape(NW, nchunks, cw)

    x_p = jnp.pad(x, ((0, NPAD - N), (0, 0)))
    zer = jnp.zeros((STRIPE, D), jnp.float32)
    b1r = b1.reshape(1, D)
    b2r = b2.reshape(1, D)
    b3r = b3.reshape(1, D)

    degp = _sc_degree(dst, zer)
    tmp1, dis = _tc_first(degp[0].reshape(NPAD, 1), degp[1].reshape(NPAD, 1),
                          x_p, W1)
    p1 = _sc_aggregate(tmp1, si3, di3)
    tmp2 = _tc_layer(p1, dis, b1r, W2)
    p2 = _sc_aggregate(tmp2, si3, di3)
    tmp3 = _tc_layer(p2, dis, b2r, W3)
    p3 = _sc_aggregate(tmp3, si3, di3)
    return _tc_final(p3, dis, b3r)


# speedup vs baseline: 1.0469x; 1.0469x over previous
import jax
import jax.numpy as jnp
from jax import lax
from jax.experimental import pallas as pl
from jax.experimental.pallas import tpu as pltpu
from jax.experimental.pallas import tpu_sc as plsc

N = 10000
D = 128
NC = 2
NS = 16
NW = NC * NS
NPAD = 10240
STRIPE = NPAD // NS


_HROWS = NPAD // 128
_HSTRIPE = _HROWS // NS


def _deg_body(d1_hbm, zer_hbm, out_hbm, di_v, h1_v, h2_v, idx_v, acc_s):
    c = lax.axis_index("c")
    s = lax.axis_index("s")
    w = c * NS + s
    ew = di_v.shape[0]
    pltpu.sync_copy(d1_hbm.at[pl.ds(w * ew, ew)], di_v)

    zv = jnp.zeros((16,), jnp.float32)

    def zfill(i, _):
        h1_v[pl.ds(i * 16, 16)] = zv
        return 0

    lax.fori_loop(0, NPAD // 16, zfill, 0)

    def ifill(k, _):
        idx_v[pl.ds(k * 16, 16)] = lax.iota(jnp.int32, 16) + k * 16
        return 0

    lax.fori_loop(0, _HROWS // 16, ifill, 0)

    @pl.when(s < _HROWS // 8)
    def _():
        pltpu.sync_copy(zer_hbm.at[pl.ds(0, 8)],
                        acc_s.at[pl.ds(s * 8, 8)])

    ones = jnp.full((16,), 1.0, jnp.float32)

    def hadd(i, _):
        dv = di_v[pl.ds(i * 16, 16)]
        plsc.addupdate_scatter(h1_v, [dv], ones)
        return 0

    lax.fori_loop(0, ew // 16, hadd, 0)

    def rsh(i, _):
        r = i // 8
        k = i - r * 8
        h2_v[r, pl.ds(k * 16, 16)] = h1_v[pl.ds(i * 16, 16)]
        return 0

    lax.fori_loop(0, NPAD // 16, rsh, 0)
    plsc.subcore_barrier()
    pltpu.sync_copy(h2_v, acc_s.at[idx_v], add=True)
    plsc.subcore_barrier()

    @pl.when(s < _HROWS // 8)
    def _():
        pltpu.sync_copy(acc_s.at[pl.ds(s * 8, 8)],
                        out_hbm.at[c].at[pl.ds(s * 8, 8)])


def _sc_degree(d1, zer):
    ew = d1.shape[0] // NW
    mesh = plsc.VectorSubcoreMesh(core_axis_name="c", subcore_axis_name="s")
    f = pl.kernel(
        _deg_body,
        out_type=jax.ShapeDtypeStruct((NC, _HROWS, 128), jnp.float32),
        mesh=mesh,
        scratch_types=[
            pltpu.VMEM((ew,), jnp.int32),
            pltpu.VMEM((NPAD,), jnp.float32),
            pltpu.VMEM((_HROWS, 128), jnp.float32),
            pltpu.VMEM((_HROWS,), jnp.int32),
            pltpu.VMEM_SHARED((_HROWS, 128), jnp.float32),
        ],
        compiler_params=pltpu.CompilerParams(needs_layout_passes=False),
    )
    return f(d1, zer)


def _agg_body(h_hbm, si_hbm, di_hbm, out0_hbm, out1_hbm,
              si_v, di_v, rows_v, rb2_v, zb_v, acc_s, sem, sem2):
    c = lax.axis_index("c")
    s = lax.axis_index("s")
    w = c * NS + s
    nchunks = si_hbm.shape[1]
    nstage = si_v.shape[0]

    pltpu.async_copy(si_hbm.at[w].at[pl.ds(0, nstage)], si_v, sem)
    pltpu.async_copy(di_hbm.at[w].at[pl.ds(0, nstage)], di_v, sem2)

    @pl.when(c == 0)
    def _():
        pltpu.sync_copy(h_hbm.at[pl.ds(s * STRIPE, STRIPE)],
                        acc_s.at[pl.ds(s * STRIPE, STRIPE)])

    @pl.when(c != 0)
    def _():
        zv = jnp.zeros((16,), jnp.float32)
        zrows = zb_v.shape[0]

        def zfill(i, _):
            r = i // 8
            k = i - r * 8
            zb_v[r, pl.ds(k * 16, 16)] = zv
            return 0

        lax.fori_loop(0, zrows * 8, zfill, 0)
        for k in range(STRIPE // zrows):
            pltpu.sync_copy(zb_v,
                            acc_s.at[pl.ds(s * STRIPE + k * zrows, zrows)])

    pltpu.make_async_copy(si_hbm.at[w].at[pl.ds(0, nstage)], si_v, sem).wait()
    pltpu.make_async_copy(di_hbm.at[w].at[pl.ds(0, nstage)], di_v, sem2).wait()
    plsc.subcore_barrier()

    for p in range(nchunks // nstage):
        if p > 0:
            pltpu.sync_copy(si_hbm.at[w].at[pl.ds(p * nstage, nstage)], si_v)
            pltpu.sync_copy(di_hbm.at[w].at[pl.ds(p * nstage, nstage)], di_v)
        pltpu.sync_copy(h_hbm.at[si_v.at[0]], rows_v)
        pltpu.async_copy(rows_v, acc_s.at[di_v.at[0]], sem, add=True)

        def body(k, _):
            j1 = 2 * k + 1
            j2 = j1 + 1
            pltpu.sync_copy(h_hbm.at[si_v.at[j1]], rb2_v)
            pltpu.async_copy(rb2_v, acc_s.at[di_v.at[j1]], sem2, add=True)
            pltpu.make_async_copy(rows_v, acc_s.at[di_v.at[j1 - 1]],
                                  sem).wait()

            @pl.when(j2 < nstage)
            def _():
                pltpu.sync_copy(h_hbm.at[si_v.at[j2]], rows_v)
                pltpu.async_copy(rows_v, acc_s.at[di_v.at[j2]], sem, add=True)

            pltpu.make_async_copy(rb2_v, acc_s.at[di_v.at[j1]], sem2).wait()
            return 0

        lax.fori_loop(0, nstage // 2, body, 0)

    plsc.subcore_barrier()

    @pl.when(c == 0)
    def _():
        pltpu.sync_copy(acc_s.at[pl.ds(s * STRIPE, STRIPE)],
                        out0_hbm.at[pl.ds(s * STRIPE, STRIPE)])

    @pl.when(c != 0)
    def _():
        pltpu.sync_copy(acc_s.at[pl.ds(s * STRIPE, STRIPE)],
                        out1_hbm.at[pl.ds(s * STRIPE, STRIPE)])


def _sc_aggregate(h, si3, di3):
    nchunks = si3.shape[1]
    cw = si3.shape[2]
    mesh = plsc.VectorSubcoreMesh(core_axis_name="c", subcore_axis_name="s")
    f = pl.kernel(
        _agg_body,
        out_type=[jax.ShapeDtypeStruct((NPAD, D), jnp.float32),
                  jax.ShapeDtypeStruct((NPAD, D), jnp.float32)],
        mesh=mesh,
        scratch_types=[
            pltpu.VMEM((nchunks // 2, cw), jnp.int32),
            pltpu.VMEM((nchunks // 2, cw), jnp.int32),
            pltpu.VMEM((cw, D), jnp.float32),
            pltpu.VMEM((cw, D), jnp.float32),
            pltpu.VMEM((40, D), jnp.float32),
            pltpu.VMEM_SHARED((NPAD, D), jnp.float32),
            pltpu.SemaphoreType.DMA,
            pltpu.SemaphoreType.DMA,
        ],
    )
    return f(h, si3, di3)


_BLK = 2048
_GRID = NPAD // _BLK


def _mm1_body(d0_ref, d1_ref, x_ref, w_ref, o_ref, dis_ref):
    deg = 1.0 + d0_ref[...] + d1_ref[...]
    dis = lax.rsqrt(deg)
    dis_ref[...] = dis
    o_ref[...] = jnp.dot(x_ref[...], w_ref[...],
                         preferred_element_type=jnp.float32) * dis


def _tc_first(d0, d1, x, w):
    return pl.pallas_call(
        _mm1_body,
        grid=(_GRID,),
        in_specs=[
            pl.BlockSpec((_BLK, 1), lambda i: (i, 0)),
            pl.BlockSpec((_BLK, 1), lambda i: (i, 0)),
            pl.BlockSpec((_BLK, D), lambda i: (i, 0)),
            pl.BlockSpec((D, D), lambda i: (0, 0)),
        ],
        out_specs=[
            pl.BlockSpec((_BLK, D), lambda i: (i, 0)),
            pl.BlockSpec((_BLK, 1), lambda i: (i, 0)),
        ],
        out_shape=[
            jax.ShapeDtypeStruct((NPAD, D), jnp.float32),
            jax.ShapeDtypeStruct((NPAD, 1), jnp.float32),
        ],
    )(d0, d1, x, w)


def _layer_body(p0_ref, p1_ref, dis_ref, b_ref, w_ref, o_ref):
    dis = dis_ref[...]
    t = (p0_ref[...] + p1_ref[...]) * dis + b_ref[...]
    a = jnp.maximum(t, 0.0)
    o_ref[...] = jnp.dot(a, w_ref[...],
                         preferred_element_type=jnp.float32) * dis


def _tc_layer(p, dis, b, w):
    return pl.pallas_call(
        _layer_body,
        grid=(_GRID,),
        in_specs=[
            pl.BlockSpec((_BLK, D), lambda i: (i, 0)),
            pl.BlockSpec((_BLK, D), lambda i: (i, 0)),
            pl.BlockSpec((_BLK, 1), lambda i: (i, 0)),
            pl.BlockSpec((1, D), lambda i: (0, 0)),
            pl.BlockSpec((D, D), lambda i: (0, 0)),
        ],
        out_specs=pl.BlockSpec((_BLK, D), lambda i: (i, 0)),
        out_shape=jax.ShapeDtypeStruct((NPAD, D), jnp.float32),
    )(p[0], p[1], dis, b, w)


def _final_body(p0_ref, p1_ref, dis_ref, b_ref, o_ref):
    t = (p0_ref[...] + p1_ref[...]) * dis_ref[...] + b_ref[...]
    m = jnp.max(t, axis=1, keepdims=True)
    e = jnp.exp(t - m)
    ssum = jnp.sum(e, axis=1, keepdims=True)
    o_ref[...] = t - m - jnp.log(ssum)


def _tc_final(p, dis, b):
    blk = 1000
    return pl.pallas_call(
        _final_body,
        grid=(N // blk,),
        in_specs=[
            pl.BlockSpec((blk, D), lambda i: (i, 0)),
            pl.BlockSpec((blk, D), lambda i: (i, 0)),
            pl.BlockSpec((blk, 1), lambda i: (i, 0)),
            pl.BlockSpec((1, D), lambda i: (0, 0)),
        ],
        out_specs=pl.BlockSpec((blk, D), lambda i: (i, 0)),
        out_shape=jax.ShapeDtypeStruct((N, D), jnp.float32),
    )(p[0], p[1], dis, b)


@jax.jit
def kernel(x, edge_index, W1, b1, W2, b2, W3, b3):
    E = edge_index.shape[1]
    nchunks = 80
    cw = E // (NW * nchunks)

    src = edge_index[0].astype(jnp.int32)
    dst = edge_index[1].astype(jnp.int32)
    si3 = src.reshape(NW, nchunks, cw)
    di3 = dst.reshape(NW, nchunks, cw)

    x_p = jnp.pad(x, ((0, NPAD - N), (0, 0)))
    zer = jnp.zeros((STRIPE, D), jnp.float32)
    b1r = b1.reshape(1, D)
    b2r = b2.reshape(1, D)
    b3r = b3.reshape(1, D)

    degp = _sc_degree(dst, zer)
    tmp1, dis = _tc_first(degp[0].reshape(NPAD, 1), degp[1].reshape(NPAD, 1),
                          x_p, W1)
    p1 = _sc_aggregate(tmp1, si3, di3)
    tmp2 = _tc_layer(p1, dis, b1r, W2)
    p2 = _sc_aggregate(tmp2, si3, di3)
    tmp3 = _tc_layer(p2, dis, b2r, W3)
    p3 = _sc_aggregate(tmp3, si3, di3)
    return _tc_final(p3, dis, b3r)


# speedup vs baseline: 1.0492x; 1.0022x over previous
import jax
import jax.numpy as jnp
from jax import lax
from jax.experimental import pallas as pl
from jax.experimental.pallas import tpu as pltpu
from jax.experimental.pallas import tpu_sc as plsc

N = 10000
D = 128
NC = 2
NS = 16
NW = NC * NS
NPAD = 10240
STRIPE = NPAD // NS


_HROWS = NPAD // 128
_HSTRIPE = _HROWS // NS


def _deg_body(d1_hbm, zer_hbm, out_hbm, di_v, h1_v, h2_v, idx_v, acc_s):
    c = lax.axis_index("c")
    s = lax.axis_index("s")
    w = c * NS + s
    ew = di_v.shape[0]
    pltpu.sync_copy(d1_hbm.at[pl.ds(w * ew, ew)], di_v)

    zv = jnp.zeros((16,), jnp.float32)

    def zfill(i, _):
        h1_v[pl.ds(i * 16, 16)] = zv
        return 0

    lax.fori_loop(0, NPAD // 16, zfill, 0)

    def ifill(k, _):
        idx_v[pl.ds(k * 16, 16)] = lax.iota(jnp.int32, 16) + k * 16
        return 0

    lax.fori_loop(0, _HROWS // 16, ifill, 0)

    @pl.when(s < _HROWS // 8)
    def _():
        pltpu.sync_copy(zer_hbm.at[pl.ds(0, 8)],
                        acc_s.at[pl.ds(s * 8, 8)])

    ones = jnp.full((16,), 1.0, jnp.float32)

    def hadd(i, _):
        dv = di_v[pl.ds(i * 16, 16)]
        plsc.addupdate_scatter(h1_v, [dv], ones)
        return 0

    lax.fori_loop(0, ew // 16, hadd, 0)

    def rsh(i, _):
        r = i // 8
        k = i - r * 8
        h2_v[r, pl.ds(k * 16, 16)] = h1_v[pl.ds(i * 16, 16)]
        return 0

    lax.fori_loop(0, NPAD // 16, rsh, 0)
    plsc.subcore_barrier()
    pltpu.sync_copy(h2_v, acc_s.at[idx_v], add=True)
    plsc.subcore_barrier()

    @pl.when(s < _HROWS // 8)
    def _():
        pltpu.sync_copy(acc_s.at[pl.ds(s * 8, 8)],
                        out_hbm.at[c].at[pl.ds(s * 8, 8)])


def _sc_degree(d1, zer):
    ew = d1.shape[0] // NW
    mesh = plsc.VectorSubcoreMesh(core_axis_name="c", subcore_axis_name="s")
    f = pl.kernel(
        _deg_body,
        out_type=jax.ShapeDtypeStruct((NC, _HROWS, 128), jnp.float32),
        mesh=mesh,
        scratch_types=[
            pltpu.VMEM((ew,), jnp.int32),
            pltpu.VMEM((NPAD,), jnp.float32),
            pltpu.VMEM((_HROWS, 128), jnp.float32),
            pltpu.VMEM((_HROWS,), jnp.int32),
            pltpu.VMEM_SHARED((_HROWS, 128), jnp.float32),
        ],
        compiler_params=pltpu.CompilerParams(needs_layout_passes=False),
    )
    return f(d1, zer)


def _agg_body(h_hbm, si_hbm, di_hbm, out0_hbm, out1_hbm,
              si_v, di_v, rows_v, rb2_v, zb_v, acc_s, sem, sem2):
    c = lax.axis_index("c")
    s = lax.axis_index("s")
    w = c * NS + s
    nchunks = si_hbm.shape[1]
    nstage = si_v.shape[0]

    pltpu.async_copy(si_hbm.at[w].at[pl.ds(0, nstage)], si_v, sem)
    pltpu.async_copy(di_hbm.at[w].at[pl.ds(0, nstage)], di_v, sem2)

    zv = jnp.zeros((16,), jnp.float32)
    zrows = zb_v.shape[0]

    def zfill(i, _):
        r = i // 8
        k = i - r * 8
        zb_v[r, pl.ds(k * 16, 16)] = zv
        return 0

    lax.fori_loop(0, zrows * 8, zfill, 0)
    for k in range(STRIPE // zrows):
        pltpu.sync_copy(zb_v,
                        acc_s.at[pl.ds(s * STRIPE + k * zrows, zrows)])

    pltpu.make_async_copy(si_hbm.at[w].at[pl.ds(0, nstage)], si_v, sem).wait()
    pltpu.make_async_copy(di_hbm.at[w].at[pl.ds(0, nstage)], di_v, sem2).wait()
    plsc.subcore_barrier()

    for p in range(nchunks // nstage):
        if p > 0:
            pltpu.sync_copy(si_hbm.at[w].at[pl.ds(p * nstage, nstage)], si_v)
            pltpu.sync_copy(di_hbm.at[w].at[pl.ds(p * nstage, nstage)], di_v)
        pltpu.sync_copy(h_hbm.at[si_v.at[0]], rows_v)
        pltpu.async_copy(rows_v, acc_s.at[di_v.at[0]], sem, add=True)

        def body(k, _):
            j1 = 2 * k + 1
            j2 = j1 + 1
            pltpu.sync_copy(h_hbm.at[si_v.at[j1]], rb2_v)
            pltpu.async_copy(rb2_v, acc_s.at[di_v.at[j1]], sem2, add=True)
            pltpu.make_async_copy(rows_v, acc_s.at[di_v.at[j1 - 1]],
                                  sem).wait()

            @pl.when(j2 < nstage)
            def _():
                pltpu.sync_copy(h_hbm.at[si_v.at[j2]], rows_v)
                pltpu.async_copy(rows_v, acc_s.at[di_v.at[j2]], sem, add=True)

            pltpu.make_async_copy(rb2_v, acc_s.at[di_v.at[j1]], sem2).wait()
            return 0

        lax.fori_loop(0, nstage // 2, body, 0)

    plsc.subcore_barrier()

    @pl.when(c == 0)
    def _():
        pltpu.sync_copy(acc_s.at[pl.ds(s * STRIPE, STRIPE)],
                        out0_hbm.at[pl.ds(s * STRIPE, STRIPE)])

    @pl.when(c != 0)
    def _():
        pltpu.sync_copy(acc_s.at[pl.ds(s * STRIPE, STRIPE)],
                        out1_hbm.at[pl.ds(s * STRIPE, STRIPE)])


def _sc_aggregate(h, si3, di3):
    nchunks = si3.shape[1]
    cw = si3.shape[2]
    mesh = plsc.VectorSubcoreMesh(core_axis_name="c", subcore_axis_name="s")
    f = pl.kernel(
        _agg_body,
        out_type=[jax.ShapeDtypeStruct((NPAD, D), jnp.float32),
                  jax.ShapeDtypeStruct((NPAD, D), jnp.float32)],
        mesh=mesh,
        scratch_types=[
            pltpu.VMEM((nchunks // 2, cw), jnp.int32),
            pltpu.VMEM((nchunks // 2, cw), jnp.int32),
            pltpu.VMEM((cw, D), jnp.float32),
            pltpu.VMEM((cw, D), jnp.float32),
            pltpu.VMEM((40, D), jnp.float32),
            pltpu.VMEM_SHARED((NPAD, D), jnp.float32),
            pltpu.SemaphoreType.DMA,
            pltpu.SemaphoreType.DMA,
        ],
    )
    return f(h, si3, di3)


_BLK = 2048
_GRID = NPAD // _BLK


def _mm1_body(d0_ref, d1_ref, x_ref, w_ref, o_ref, dis_ref):
    deg = 1.0 + d0_ref[...] + d1_ref[...]
    dis = lax.rsqrt(deg)
    dis_ref[...] = dis
    o_ref[...] = jnp.dot(x_ref[...], w_ref[...],
                         preferred_element_type=jnp.float32) * dis


def _tc_first(d0, d1, x, w):
    return pl.pallas_call(
        _mm1_body,
        grid=(_GRID,),
        in_specs=[
            pl.BlockSpec((_BLK, 1), lambda i: (i, 0)),
            pl.BlockSpec((_BLK, 1), lambda i: (i, 0)),
            pl.BlockSpec((_BLK, D), lambda i: (i, 0)),
            pl.BlockSpec((D, D), lambda i: (0, 0)),
        ],
        out_specs=[
            pl.BlockSpec((_BLK, D), lambda i: (i, 0)),
            pl.BlockSpec((_BLK, 1), lambda i: (i, 0)),
        ],
        out_shape=[
            jax.ShapeDtypeStruct((NPAD, D), jnp.float32),
            jax.ShapeDtypeStruct((NPAD, 1), jnp.float32),
        ],
    )(d0, d1, x, w)


def _layer_body(p0_ref, p1_ref, t_ref, dis_ref, b_ref, w_ref, o_ref):
    dis = dis_ref[...]
    t = (p0_ref[...] + p1_ref[...] + t_ref[...]) * dis + b_ref[...]
    a = jnp.maximum(t, 0.0)
    o_ref[...] = jnp.dot(a, w_ref[...],
                         preferred_element_type=jnp.float32) * dis


def _tc_layer(p, tmp, dis, b, w):
    return pl.pallas_call(
        _layer_body,
        grid=(_GRID,),
        in_specs=[
            pl.BlockSpec((_BLK, D), lambda i: (i, 0)),
            pl.BlockSpec((_BLK, D), lambda i: (i, 0)),
            pl.BlockSpec((_BLK, D), lambda i: (i, 0)),
            pl.BlockSpec((_BLK, 1), lambda i: (i, 0)),
            pl.BlockSpec((1, D), lambda i: (0, 0)),
            pl.BlockSpec((D, D), lambda i: (0, 0)),
        ],
        out_specs=pl.BlockSpec((_BLK, D), lambda i: (i, 0)),
        out_shape=jax.ShapeDtypeStruct((NPAD, D), jnp.float32),
    )(p[0], p[1], tmp, dis, b, w)


def _final_body(p0_ref, p1_ref, t_ref, dis_ref, b_ref, o_ref):
    t = (p0_ref[...] + p1_ref[...] + t_ref[...]) * dis_ref[...] + b_ref[...]
    m = jnp.max(t, axis=1, keepdims=True)
    e = jnp.exp(t - m)
    ssum = jnp.sum(e, axis=1, keepdims=True)
    o_ref[...] = t - m - jnp.log(ssum)


def _tc_final(p, tmp, dis, b):
    blk = 1000
    return pl.pallas_call(
        _final_body,
        grid=(N // blk,),
        in_specs=[
            pl.BlockSpec((blk, D), lambda i: (i, 0)),
            pl.BlockSpec((blk, D), lambda i: (i, 0)),
            pl.BlockSpec((blk, D), lambda i: (i, 0)),
            pl.BlockSpec((blk, 1), lambda i: (i, 0)),
            pl.BlockSpec((1, D), lambda i: (0, 0)),
        ],
        out_specs=pl.BlockSpec((blk, D), lambda i: (i, 0)),
        out_shape=jax.ShapeDtypeStruct((N, D), jnp.float32),
    )(p[0], p[1], tmp, dis, b)


@jax.jit
def kernel(x, edge_index, W1, b1, W2, b2, W3, b3):
    E = edge_index.shape[1]
    nchunks = 80
    cw = E // (NW * nchunks)

    src = edge_index[0].astype(jnp.int32)
    dst = edge_index[1].astype(jnp.int32)
    si3 = src.reshape(NW, nchunks, cw)
    di3 = dst.reshape(NW, nchunks, cw)

    x_p = jnp.pad(x, ((0, NPAD - N), (0, 0)))
    zer = jnp.zeros((STRIPE, D), jnp.float32)
    b1r = b1.reshape(1, D)
    b2r = b2.reshape(1, D)
    b3r = b3.reshape(1, D)

    degp = _sc_degree(dst, zer)
    tmp1, dis = _tc_first(degp[0].reshape(NPAD, 1), degp[1].reshape(NPAD, 1),
                          x_p, W1)
    p1 = _sc_aggregate(tmp1, si3, di3)
    tmp2 = _tc_layer(p1, tmp1, dis, b1r, W2)
    p2 = _sc_aggregate(tmp2, si3, di3)
    tmp3 = _tc_layer(p2, tmp2, dis, b2r, W3)
    p3 = _sc_aggregate(tmp3, si3, di3)
    return _tc_final(p3, tmp3, dis, b3r)
